# baseline (device time: 327771 ns/iter reference)
import jax
import jax.numpy as jnp
from jax import lax
from jax.experimental import pallas as pl
from jax.experimental.pallas import tpu as pltpu

N_DEV = 32


def kernel(A, B):
    m, k = A.shape
    _, n = B.shape
    rows = m // N_DEV

    def body(a_ref, b_ref, out_ref, p_ref, sbuf_ref, rs_recv_ref,
             send_sems, rs_recv_sems, ag_recv_sems):
        my = lax.axis_index("i")
        left = (my - 1) % N_DEV
        right = (my + 1) % N_DEV

        barrier_sem = pltpu.get_barrier_semaphore()
        for nbr in (left, right):
            pl.semaphore_signal(
                barrier_sem, inc=1,
                device_id=(nbr,), device_id_type=pl.DeviceIdType.MESH,
            )
        pl.semaphore_wait(barrier_sem, 2)

        p_ref[...] = jnp.dot(
            a_ref[...], b_ref[...], preferred_element_type=jnp.float32
        )

        for s in range(N_DEV - 1):
            idx = (my - s) % N_DEV
            chunk = p_ref[pl.ds(idx * rows, rows), :]
            if s == 0:
                val = chunk
            else:
                val = chunk + rs_recv_ref[s - 1]
            sbuf_ref[s % 2, :, :] = val
            rdma = pltpu.make_async_remote_copy(
                src_ref=sbuf_ref.at[s % 2],
                dst_ref=rs_recv_ref.at[s],
                send_sem=send_sems.at[s % 2],
                recv_sem=rs_recv_sems.at[s],
                device_id=(right,),
                device_id_type=pl.DeviceIdType.MESH,
            )
            rdma.start()
            rdma.wait_send()
            rdma.wait_recv()

        owned = (my + 1) % N_DEV
        final = p_ref[pl.ds(owned * rows, rows), :] + rs_recv_ref[N_DEV - 2]
        out_ref[pl.ds(owned * rows, rows), :] = jnp.maximum(final, 0.0)

        for s in range(N_DEV - 1):
            send_idx = (my + 1 - s) % N_DEV
            rdma = pltpu.make_async_remote_copy(
                src_ref=out_ref.at[pl.ds(send_idx * rows, rows), :],
                dst_ref=out_ref.at[pl.ds(send_idx * rows, rows), :],
                send_sem=send_sems.at[s % 2],
                recv_sem=ag_recv_sems.at[s],
                device_id=(right,),
                device_id_type=pl.DeviceIdType.MESH,
            )
            rdma.start()
            rdma.wait_send()
            rdma.wait_recv()

    return pl.pallas_call(
        body,
        out_shape=jax.ShapeDtypeStruct((m, n), jnp.float32),
        in_specs=[
            pl.BlockSpec(memory_space=pltpu.VMEM),
            pl.BlockSpec(memory_space=pltpu.VMEM),
        ],
        out_specs=pl.BlockSpec(memory_space=pltpu.VMEM),
        scratch_shapes=[
            pltpu.VMEM((m, n), jnp.float32),
            pltpu.VMEM((2, rows, n), jnp.float32),
            pltpu.VMEM((N_DEV - 1, rows, n), jnp.float32),
            pltpu.SemaphoreType.DMA((2,)),
            pltpu.SemaphoreType.DMA((N_DEV - 1,)),
            pltpu.SemaphoreType.DMA((N_DEV - 1,)),
        ],
        compiler_params=pltpu.CompilerParams(collective_id=0),
    )(A, B)


# device time: 159063 ns/iter; 2.0606x vs baseline; 2.0606x over previous
import jax
import jax.numpy as jnp
from jax import lax
from jax.experimental import pallas as pl
from jax.experimental.pallas import tpu as pltpu

N_DEV = 32
NP = 8
NZ = 4

_RING = [0, 1, 2, 5, 6, 7, 4, 3]
_R_OF = [0, 1, 2, 7, 6, 3, 4, 5]
_RIGHT_M = [_RING[(_R_OF[k] + 1) % NP] for k in range(NP)]
_LEFT_M = [_RING[(_R_OF[k] - 1) % NP] for k in range(NP)]


def _lut(idx, table):
    out = jnp.int32(0)
    for k, v in enumerate(table):
        out = out + jnp.where(idx == k, jnp.int32(v), jnp.int32(0))
    return out


def kernel(A, B):
    m_rows, k = A.shape
    _, n = B.shape
    BLK = m_rows // NP
    HALF = BLK // 2
    SLIV = HALF // NZ

    def body(a_ref, b_ref, out_ref, p_ref,
             sbuf_cw, sbuf_ccw, rs_recv_cw, rs_recv_ccw,
             zbuf_cw, zbuf_ccw, zrecv_cw, zrecv_ccw,
             ssem_cw, ssem_ccw, zs_cw, zs_ccw,
             rsem_cw, rsem_ccw, zr_cw, zr_ccw,
             agz_cw, agz_ccw, aga_cw, aga_ccw):
        my = lax.axis_index("i")
        q = my // NP
        m = my % NP
        r = _lut(m, _R_OF)
        right_dev = q * NP + _lut(m, _RIGHT_M)
        left_dev = q * NP + _lut(m, _LEFT_M)
        up_dev = ((q + 1) % NZ) * NP + m
        down_dev = ((q - 1) % NZ) * NP + m

        barrier_sem = pltpu.get_barrier_semaphore()
        for nbr in (left_dev, right_dev, up_dev, down_dev):
            pl.semaphore_signal(
                barrier_sem, inc=1,
                device_id=(nbr,), device_id_type=pl.DeviceIdType.MESH,
            )
        pl.semaphore_wait(barrier_sem, 4)

        def mm(row0, nrows):
            p_ref[pl.ds(row0, nrows), :] = jnp.dot(
                a_ref[pl.ds(row0, nrows), :], b_ref[...],
                preferred_element_type=jnp.float32,
            )

        mm(r * BLK, BLK)

        for s in range(NP - 1):
            idx_cw = (r - s) % NP
            idx_ccw = (r + s) % NP
            val_cw = p_ref[pl.ds(idx_cw * BLK, HALF), :]
            val_ccw = p_ref[pl.ds(idx_ccw * BLK + HALF, HALF), :]
            if s > 0:
                val_cw = val_cw + rs_recv_cw[s - 1]
                val_ccw = val_ccw + rs_recv_ccw[s - 1]
            sbuf_cw[s % 2, :, :] = val_cw
            sbuf_ccw[s % 2, :, :] = val_ccw
            rd_cw = pltpu.make_async_remote_copy(
                src_ref=sbuf_cw.at[s % 2], dst_ref=rs_recv_cw.at[s],
                send_sem=ssem_cw.at[s % 2], recv_sem=rsem_cw.at[s],
                device_id=(right_dev,), device_id_type=pl.DeviceIdType.MESH,
            )
            rd_ccw = pltpu.make_async_remote_copy(
                src_ref=sbuf_ccw.at[s % 2], dst_ref=rs_recv_ccw.at[s],
                send_sem=ssem_ccw.at[s % 2], recv_sem=rsem_ccw.at[s],
                device_id=(left_dev,), device_id_type=pl.DeviceIdType.MESH,
            )
            rd_cw.start()
            rd_ccw.start()
            mm(((r - s - 1) % NP) * BLK, HALF)
            mm(((r + s + 1) % NP) * BLK + HALF, HALF)
            rd_cw.wait_send()
            rd_ccw.wait_send()
            rd_cw.wait_recv()
            rd_ccw.wait_recv()

        base_cw = ((r + 1) % NP) * BLK
        base_ccw = ((r - 1) % NP) * BLK + HALF
        p_ref[pl.ds(base_cw, HALF), :] = (
            p_ref[pl.ds(base_cw, HALF), :] + rs_recv_cw[NP - 2]
        )
        p_ref[pl.ds(base_ccw, HALF), :] = (
            p_ref[pl.ds(base_ccw, HALF), :] + rs_recv_ccw[NP - 2]
        )

        for s in range(NZ - 1):
            j_cw = (q - s) % NZ
            j_ccw = (q + s) % NZ
            val_cw = p_ref[pl.ds(base_cw + j_cw * SLIV, SLIV), :]
            val_ccw = p_ref[pl.ds(base_ccw + j_ccw * SLIV, SLIV), :]
            if s > 0:
                val_cw = val_cw + zrecv_cw[s - 1]
                val_ccw = val_ccw + zrecv_ccw[s - 1]
            zbuf_cw[s % 2, :, :] = val_cw
            zbuf_ccw[s % 2, :, :] = val_ccw
            rd_cw = pltpu.make_async_remote_copy(
                src_ref=zbuf_cw.at[s % 2], dst_ref=zrecv_cw.at[s],
                send_sem=zs_cw.at[s % 2], recv_sem=zr_cw.at[s],
                device_id=(up_dev,), device_id_type=pl.DeviceIdType.MESH,
            )
            rd_ccw = pltpu.make_async_remote_copy(
                src_ref=zbuf_ccw.at[s % 2], dst_ref=zrecv_ccw.at[s],
                send_sem=zs_ccw.at[s % 2], recv_sem=zr_ccw.at[s],
                device_id=(down_dev,), device_id_type=pl.DeviceIdType.MESH,
            )
            rd_cw.start()
            rd_ccw.start()
            rd_cw.wait_send()
            rd_ccw.wait_send()
            rd_cw.wait_recv()
            rd_ccw.wait_recv()

        j_own_cw = (q + 1) % NZ
        j_own_ccw = (q - 1) % NZ
        row_cw = base_cw + j_own_cw * SLIV
        row_ccw = base_ccw + j_own_ccw * SLIV
        out_ref[pl.ds(row_cw, SLIV), :] = jnp.maximum(
            p_ref[pl.ds(row_cw, SLIV), :] + zrecv_cw[NZ - 2], 0.0
        )
        out_ref[pl.ds(row_ccw, SLIV), :] = jnp.maximum(
            p_ref[pl.ds(row_ccw, SLIV), :] + zrecv_ccw[NZ - 2], 0.0
        )

        for s in range(NZ - 1):
            j_cw = (q + 1 - s) % NZ
            j_ccw = (q - 1 + s) % NZ
            rd_cw = pltpu.make_async_remote_copy(
                src_ref=out_ref.at[pl.ds(base_cw + j_cw * SLIV, SLIV), :],
                dst_ref=out_ref.at[pl.ds(base_cw + j_cw * SLIV, SLIV), :],
                send_sem=zs_cw.at[s % 2], recv_sem=agz_cw.at[s],
                device_id=(up_dev,), device_id_type=pl.DeviceIdType.MESH,
            )
            rd_ccw = pltpu.make_async_remote_copy(
                src_ref=out_ref.at[pl.ds(base_ccw + j_ccw * SLIV, SLIV), :],
                dst_ref=out_ref.at[pl.ds(base_ccw + j_ccw * SLIV, SLIV), :],
                send_sem=zs_ccw.at[s % 2], recv_sem=agz_ccw.at[s],
                device_id=(down_dev,), device_id_type=pl.DeviceIdType.MESH,
            )
            rd_cw.start()
            rd_ccw.start()
            rd_cw.wait_send()
            rd_ccw.wait_send()
            rd_cw.wait_recv()
            rd_ccw.wait_recv()

        for s in range(NP - 1):
            c_cw = (r + 1 - s) % NP
            c_ccw = (r - 1 + s) % NP
            rd_cw = pltpu.make_async_remote_copy(
                src_ref=out_ref.at[pl.ds(c_cw * BLK, HALF), :],
                dst_ref=out_ref.at[pl.ds(c_cw * BLK, HALF), :],
                send_sem=ssem_cw.at[s % 2], recv_sem=aga_cw.at[s],
                device_id=(right_dev,), device_id_type=pl.DeviceIdType.MESH,
            )
            rd_ccw = pltpu.make_async_remote_copy(
                src_ref=out_ref.at[pl.ds(c_ccw * BLK + HALF, HALF), :],
                dst_ref=out_ref.at[pl.ds(c_ccw * BLK + HALF, HALF), :],
                send_sem=ssem_ccw.at[s % 2], recv_sem=aga_ccw.at[s],
                device_id=(left_dev,), device_id_type=pl.DeviceIdType.MESH,
            )
            rd_cw.start()
            rd_ccw.start()
            rd_cw.wait_send()
            rd_ccw.wait_send()
            rd_cw.wait_recv()
            rd_ccw.wait_recv()

    return pl.pallas_call(
        body,
        out_shape=jax.ShapeDtypeStruct((m_rows, n), jnp.float32),
        in_specs=[
            pl.BlockSpec(memory_space=pltpu.VMEM),
            pl.BlockSpec(memory_space=pltpu.VMEM),
        ],
        out_specs=pl.BlockSpec(memory_space=pltpu.VMEM),
        scratch_shapes=[
            pltpu.VMEM((m_rows, n), jnp.float32),
            pltpu.VMEM((2, HALF, n), jnp.float32),
            pltpu.VMEM((2, HALF, n), jnp.float32),
            pltpu.VMEM((NP - 1, HALF, n), jnp.float32),
            pltpu.VMEM((NP - 1, HALF, n), jnp.float32),
            pltpu.VMEM((2, SLIV, n), jnp.float32),
            pltpu.VMEM((2, SLIV, n), jnp.float32),
            pltpu.VMEM((NZ - 1, SLIV, n), jnp.float32),
            pltpu.VMEM((NZ - 1, SLIV, n), jnp.float32),
            pltpu.SemaphoreType.DMA((2,)),
            pltpu.SemaphoreType.DMA((2,)),
            pltpu.SemaphoreType.DMA((2,)),
            pltpu.SemaphoreType.DMA((2,)),
            pltpu.SemaphoreType.DMA((NP - 1,)),
            pltpu.SemaphoreType.DMA((NP - 1,)),
            pltpu.SemaphoreType.DMA((NZ - 1,)),
            pltpu.SemaphoreType.DMA((NZ - 1,)),
            pltpu.SemaphoreType.DMA((NZ - 1,)),
            pltpu.SemaphoreType.DMA((NZ - 1,)),
            pltpu.SemaphoreType.DMA((NP - 1,)),
            pltpu.SemaphoreType.DMA((NP - 1,)),
        ],
        compiler_params=pltpu.CompilerParams(collective_id=0),
    )(A, B)
